# baseline (speedup 1.0000x reference)
import functools
import numpy as np
import jax
import jax.numpy as jnp
from jax import lax
from jax.experimental import pallas as pl
from jax.experimental.pallas import tpu as pltpu
from jax.experimental.pallas import tpu_sc as plsc

LQ = 577
MRP = 14
NU = 64
NT = 2 * MRP + 2
N = LQ * LQ

U = 8
NUNIT = N // U
UW = U * NU

NC, NS, L = 2, 16, 16
NW = NC * NS
C = 112

WCNT_LO = 8 * ((NUNIT // 8) // NW)
NBIG = (NUNIT // 8) % NW
MPW = -(-(WCNT_LO + 8) // C)
BLK = MPW * C
PADN = WCNT_LO * (NW - 1) + 8 * NBIG + BLK


def _static_geometry():
    n = np.arange(N, dtype=np.int64)
    i, j = n // LQ, n % LQ
    im1 = np.maximum(i - 1, 0)
    jm1 = np.maximum(j - 1, 0)
    qiv, qih = im1 // 24, im1 % 24
    qjv, qjh = jm1 // 24, jm1 % 24
    cv = np.clip(qjv - qiv, -MRP, MRP) + MRP + 1
    ch = np.clip(qjh - qih, -MRP, MRP) + MRP + 1
    idx = np.where((i >= 1) & (j >= 1), cv * NT + ch, 0)
    sig_all = idx[: NUNIT * U].reshape(NUNIT, U)
    sigs, inv = np.unique(sig_all, axis=0, return_inverse=True)
    inv = inv.astype(np.int32).reshape(-1)
    tail_idx = int(idx[N - 1])
    tail_unit = int(np.argmax(sigs[:, 0] == tail_idx))
    assert sigs[tail_unit, 0] == tail_idx
    inv_pad = np.zeros((PADN,), np.int32)
    inv_pad[:NUNIT] = inv
    return sigs, inv_pad, tail_unit


def _make_table(sigs, tv, th):
    k = sigs.shape[0]
    flat = sigs.reshape(-1)
    ohv = np.zeros((k * U, NT), np.float32)
    ohh = np.zeros((k * U, NT), np.float32)
    ohv[np.arange(k * U), flat // NT] = 1.0
    ohh[np.arange(k * U), flat % NT] = 1.0

    def body(jv_ref, jh_ref, tv_ref, th_ref, s_ref):
        s_ref[...] = (jnp.dot(jv_ref[...], tv_ref[...],
                              preferred_element_type=jnp.float32,
                              precision=jax.lax.Precision.HIGHEST)
                      + jnp.dot(jh_ref[...], th_ref[...],
                                preferred_element_type=jnp.float32,
                                precision=jax.lax.Precision.HIGHEST))

    blk = k * U // 8
    return pl.pallas_call(
        body,
        grid=(8,),
        in_specs=[
            pl.BlockSpec((blk, NT), lambda g: (g, 0)),
            pl.BlockSpec((blk, NT), lambda g: (g, 0)),
            pl.BlockSpec((NT, NU), lambda g: (0, 0)),
            pl.BlockSpec((NT, NU), lambda g: (0, 0)),
        ],
        out_specs=pl.BlockSpec((blk, NU), lambda g: (g, 0)),
        out_shape=jax.ShapeDtypeStruct((k * U, NU), jnp.float32),
    )(jnp.asarray(ohv), jnp.asarray(ohh), tv, th)


def _sc_body(sp_hbm, inv_hbm, out_hbm, idxblk, buf_a, buf_b,
             gsem, wsem_a, wsem_b):
    wid = lax.axis_index("s") * NC + lax.axis_index("c")
    wbase = WCNT_LO * wid + 8 * jnp.minimum(wid, NBIG)
    wcnt = WCNT_LO + 8 * jnp.where(wid < NBIG, 1, 0)

    pltpu.sync_copy(inv_hbm.at[pl.ds(wbase, BLK)], idxblk)

    def do_chunk(m, buf, wsem):
        @pl.when(m < -(-wcnt // C))
        def _():
            off = pl.multiple_of(jnp.minimum(m * C, wcnt - C), 8)

            @pl.when(m >= 2)
            def _():
                pltpu.make_async_copy(buf, out_hbm.at[pl.ds(0, C)],
                                      wsem).wait()
            pltpu.async_copy(sp_hbm.at[idxblk.at[pl.ds(off, C)]],
                             buf, gsem).wait()
            pltpu.async_copy(buf, out_hbm.at[pl.ds(wbase + off, C)], wsem)

    def step(g, _):
        do_chunk(2 * g, buf_a, wsem_a)
        do_chunk(2 * g + 1, buf_b, wsem_b)
        return 0

    lax.fori_loop(0, (MPW + 1) // 2, step, 0)

    pltpu.make_async_copy(buf_a, out_hbm.at[pl.ds(0, C)], wsem_a).wait()
    pltpu.make_async_copy(buf_b, out_hbm.at[pl.ds(0, C)], wsem_b).wait()


def kernel(length_q, length_k, sample_embeddings_table_v, sample_embeddings_table_h):
    zero = (length_q - LQ) + (length_k - LQ)
    sigs, inv_pad, tail_unit = _static_geometry()
    shift = jnp.clip(jnp.arange(NT) + zero, 0, NT - 1)
    tv = jnp.take(sample_embeddings_table_v, shift, axis=0)
    th = jnp.take(sample_embeddings_table_h, shift, axis=0)
    sp = _make_table(sigs, tv, th)
    inv = jnp.asarray(inv_pad)

    mesh = plsc.VectorSubcoreMesh(core_axis_name="c", subcore_axis_name="s")
    out = pl.kernel(
        _sc_body,
        out_type=jax.ShapeDtypeStruct((NUNIT, UW), jnp.float32),
        mesh=mesh,
        scratch_types=[
            pltpu.VMEM((BLK,), jnp.int32),
            pltpu.VMEM((C, UW), jnp.float32),
            pltpu.VMEM((C, UW), jnp.float32),
            pltpu.SemaphoreType.DMA,
            pltpu.SemaphoreType.DMA,
            pltpu.SemaphoreType.DMA,
        ],
    )(sp.reshape(sp.shape[0] // U, UW), inv)

    tail = sp[tail_unit * U]
    flat = jnp.concatenate([out.reshape(-1), tail], axis=0)
    return flat.reshape(LQ, LQ, NU)


# speedup vs baseline: 1.5759x; 1.5759x over previous
import functools
import jax
import jax.numpy as jnp
from jax import lax
from jax.experimental import pallas as pl
from jax.experimental.pallas import tpu as pltpu
from jax.experimental.pallas import tpu_sc as plsc

LQ = 577
MRP = 14
NU = 64
NT = 2 * MRP + 2

NC, NS, L = 2, 16, 16
NW = NC * NS
TG = -(-LQ // L)
LP = TG * L


def _combine_body(tv_ref, th_ref, s_ref):
    s_ref[...] = tv_ref[...][:, None, :] + th_ref[...][None, :, :]


def _make_combined(tv, th):
    out3 = pl.pallas_call(
        _combine_body,
        out_shape=jax.ShapeDtypeStruct((NT, NT, NU), jnp.float32),
    )(tv, th)
    return out3.reshape(NT * NT, NU)


def _divmod_pos(n, d):
    q = (n.astype(jnp.float32) * (1.0 / d)).astype(jnp.int32)
    r = n - q * d
    q = jnp.where(r < 0, q - 1, q)
    q = jnp.where(r >= d, q + 1, q)
    r = n - q * d
    return q, r


def _row_indices(zero, i, qjv_ref, qjh_ref, jpos_ref, idx_ref):
    ivec = jnp.full((L,), i, jnp.int32)
    im1 = jnp.maximum(ivec - 1, 0)
    qiv, qih = _divmod_pos(im1, 24)
    ipos = jnp.where(ivec >= 1, 1, 0)
    for t in range(TG):
        qjv = qjv_ref[pl.ds(t * L, L)]
        qjh = qjh_ref[pl.ds(t * L, L)]
        jpos = jpos_ref[pl.ds(t * L, L)]
        cv = jnp.clip(qjv - qiv, -MRP, MRP) + (MRP + 1)
        ch = jnp.clip(qjh - qih, -MRP, MRP) + (MRP + 1)
        idx = (cv * NT + ch) * jpos * ipos + zero * (NT + 1)
        idx_ref[pl.ds(t * L, L)] = idx


def _sc_body(zero, s_hbm, out_hbm, qjv_ref, qjh_ref, jpos_ref, idx_ref,
             rows_a, rows_b, s_shared, gsem, wsem_a, wsem_b):
    wid = lax.axis_index("s") * NC + lax.axis_index("c")

    @pl.when(lax.axis_index("s") == 0)
    def _stage():
        pltpu.sync_copy(s_hbm, s_shared)
    plsc.subcore_barrier()

    for t in range(TG):
        j = t * L + lax.iota(jnp.int32, L)
        jm1 = jnp.maximum(j - 1, 0)
        qjv, qjh = _divmod_pos(jm1, 24)
        qjv_ref[pl.ds(t * L, L)] = qjv
        qjh_ref[pl.ds(t * L, L)] = qjh
        jpos_ref[pl.ds(t * L, L)] = jnp.where(j >= 1, 1, 0)

    def do_row(g, i, rows_ref, wsem):
        @pl.when(i < LQ)
        def _():
            @pl.when(g > 0)
            def _():
                pltpu.make_async_copy(rows_ref, out_hbm.at[i], wsem).wait()
            _row_indices(zero, i, qjv_ref, qjh_ref, jpos_ref, idx_ref)
            pltpu.async_copy(s_shared.at[idx_ref.at[pl.ds(0, LQ)]],
                             rows_ref, gsem).wait()
            pltpu.async_copy(rows_ref, out_hbm.at[i], wsem)

    def step(g, _):
        do_row(g, wid + 2 * NW * g, rows_a, wsem_a)
        do_row(g, wid + 2 * NW * g + NW, rows_b, wsem_b)
        return 0

    lax.fori_loop(0, -(-LQ // (2 * NW)), step, 0)

    pltpu.make_async_copy(rows_a, out_hbm.at[0], wsem_a).wait()
    pltpu.make_async_copy(rows_b, out_hbm.at[0], wsem_b).wait()


def kernel(length_q, length_k, sample_embeddings_table_v, sample_embeddings_table_h):
    zero = (length_q - LQ) + (length_k - LQ)
    s = _make_combined(sample_embeddings_table_v, sample_embeddings_table_h)

    mesh = plsc.VectorSubcoreMesh(core_axis_name="c", subcore_axis_name="s")
    out = pl.kernel(
        functools.partial(_sc_body, zero),
        out_type=jax.ShapeDtypeStruct((LQ, LQ, NU), jnp.float32),
        mesh=mesh,
        compiler_params=pltpu.CompilerParams(use_tc_tiling_on_sc=False),
        scratch_types=[
            pltpu.VMEM((LP,), jnp.int32),
            pltpu.VMEM((LP,), jnp.int32),
            pltpu.VMEM((LP,), jnp.int32),
            pltpu.VMEM((LP,), jnp.int32),
            pltpu.VMEM((LQ, NU), jnp.float32),
            pltpu.VMEM((LQ, NU), jnp.float32),
            pltpu.VMEM_SHARED((NT * NT, NU), jnp.float32),
            pltpu.SemaphoreType.DMA,
            pltpu.SemaphoreType.DMA,
            pltpu.SemaphoreType.DMA,
        ],
    )(s)
    return out


# speedup vs baseline: 1.5761x; 1.0001x over previous
import functools
import jax
import jax.numpy as jnp
from jax import lax
from jax.experimental import pallas as pl
from jax.experimental.pallas import tpu as pltpu
from jax.experimental.pallas import tpu_sc as plsc

LQ = 577
MRP = 14
NU = 64
NT = 2 * MRP + 2

NC, NS, L = 2, 16, 16
NW = NC * NS
TG = -(-LQ // L)
LP = TG * L


def _combine_body(tv_ref, th_ref, s_ref):
    s_ref[...] = tv_ref[...][:, None, :] + th_ref[...][None, :, :]


def _make_combined(tv, th):
    out3 = pl.pallas_call(
        _combine_body,
        out_shape=jax.ShapeDtypeStruct((NT, NT, NU), jnp.float32),
    )(tv, th)
    return out3.reshape(NT * NT, NU)


def _divmod_pos(n, d):
    q = (n.astype(jnp.float32) * (1.0 / d)).astype(jnp.int32)
    r = n - q * d
    q = jnp.where(r < 0, q - 1, q)
    q = jnp.where(r >= d, q + 1, q)
    r = n - q * d
    return q, r


def _row_indices(zero, i, qjv_ref, qjh_ref, jpos_ref, idx_ref):
    ivec = jnp.full((L,), i, jnp.int32)
    im1 = jnp.maximum(ivec - 1, 0)
    qiv, qih = _divmod_pos(im1, 24)
    ipos = jnp.where(ivec >= 1, 1, 0)
    for t in range(TG):
        qjv = qjv_ref[pl.ds(t * L, L)]
        qjh = qjh_ref[pl.ds(t * L, L)]
        jpos = jpos_ref[pl.ds(t * L, L)]
        cv = jnp.clip(qjv - qiv, -MRP, MRP) + (MRP + 1)
        ch = jnp.clip(qjh - qih, -MRP, MRP) + (MRP + 1)
        idx = (cv * NT + ch) * jpos * ipos + zero * (NT + 1)
        idx_ref[pl.ds(t * L, L)] = idx


def _sc_body(zero, s_hbm, out_hbm, qjv_ref, qjh_ref, jpos_ref, idx_ref,
             rows_a, rows_b, s_shared, gsem, wsem_a, wsem_b):
    wid = lax.axis_index("s") * NC + lax.axis_index("c")

    @pl.when(lax.axis_index("s") == 0)
    def _stage():
        pltpu.sync_copy(s_hbm, s_shared)
    plsc.subcore_barrier()

    for t in range(TG):
        j = t * L + lax.iota(jnp.int32, L)
        jm1 = jnp.maximum(j - 1, 0)
        qjv, qjh = _divmod_pos(jm1, 24)
        qjv_ref[pl.ds(t * L, L)] = qjv
        qjh_ref[pl.ds(t * L, L)] = qjh
        jpos_ref[pl.ds(t * L, L)] = jnp.where(j >= 1, 1, 0)

    def do_row(g, i, rows_ref, wsem):
        @pl.when(i < LQ)
        def _():
            @pl.when(g > 0)
            def _():
                pltpu.make_async_copy(
                    rows_ref, out_hbm.at[pl.ds(i * LQ, LQ)], wsem).wait()
            _row_indices(zero, i, qjv_ref, qjh_ref, jpos_ref, idx_ref)
            pltpu.async_copy(s_shared.at[idx_ref.at[pl.ds(0, LQ)]],
                             rows_ref, gsem).wait()
            pltpu.async_copy(rows_ref, out_hbm.at[pl.ds(i * LQ, LQ)], wsem)

    def step(g, _):
        do_row(g, wid + 2 * NW * g, rows_a, wsem_a)
        do_row(g, wid + 2 * NW * g + NW, rows_b, wsem_b)
        return 0

    lax.fori_loop(0, -(-LQ // (2 * NW)), step, 0)

    pltpu.make_async_copy(rows_a, out_hbm.at[pl.ds(0, LQ)], wsem_a).wait()
    pltpu.make_async_copy(rows_b, out_hbm.at[pl.ds(0, LQ)], wsem_b).wait()


def kernel(length_q, length_k, sample_embeddings_table_v, sample_embeddings_table_h):
    zero = (length_q - LQ) + (length_k - LQ)
    s = _make_combined(sample_embeddings_table_v, sample_embeddings_table_h)

    mesh = plsc.VectorSubcoreMesh(core_axis_name="c", subcore_axis_name="s")
    out = pl.kernel(
        functools.partial(_sc_body, zero),
        out_type=jax.ShapeDtypeStruct((LQ * LQ, NU), jnp.float32),
        mesh=mesh,
        compiler_params=pltpu.CompilerParams(use_tc_tiling_on_sc=False),
        scratch_types=[
            pltpu.VMEM((LP,), jnp.int32),
            pltpu.VMEM((LP,), jnp.int32),
            pltpu.VMEM((LP,), jnp.int32),
            pltpu.VMEM((LP,), jnp.int32),
            pltpu.VMEM((LQ, NU), jnp.float32),
            pltpu.VMEM((LQ, NU), jnp.float32),
            pltpu.VMEM_SHARED((NT * NT, NU), jnp.float32),
            pltpu.SemaphoreType.DMA,
            pltpu.SemaphoreType.DMA,
            pltpu.SemaphoreType.DMA,
        ],
    )(s)
    return out.reshape(LQ, LQ, NU)
